# baseline (speedup 1.0000x reference)
import functools

import jax
import jax.numpy as jnp
from jax import lax
from jax.experimental import pallas as pl
from jax.experimental.pallas import tpu as pltpu
from jax.experimental.pallas import tpu_sc as plsc

K = 5

_SC_MESH = plsc.VectorSubcoreMesh(core_axis_name="c", subcore_axis_name="s")


@functools.partial(
    pl.kernel,
    mesh=_SC_MESH,
    out_type=jax.ShapeDtypeStruct((16,), jnp.float32),
    scratch_types=[pltpu.VMEM((16,), jnp.float32)],
)
def _sc_probe(x_hbm, o_hbm, buf):
    @pl.when((lax.axis_index("c") == 0) & (lax.axis_index("s") == 0))
    def _():
        pltpu.sync_copy(x_hbm, buf)
        buf[...] = buf[...] + 1.0
        pltpu.sync_copy(buf, o_hbm)


def _basis_widx(pseudo):
    p = jnp.clip(pseudo, 0.0, 1.0) * (K - 1)
    lo = jnp.floor(p)
    frac = p - lo
    lo_i = jnp.clip(lo.astype(jnp.int32), 0, K - 1)
    hi_i = jnp.clip(lo_i + 1, 0, K - 1)
    b0 = 1.0 - frac
    b1 = frac
    basis = jnp.stack([b0[:, 0] * b0[:, 1], b1[:, 0] * b0[:, 1], b0[:, 0] * b1[:, 1], b1[:, 0] * b1[:, 1]], axis=1)
    widx = jnp.stack([lo_i[:, 0] + K * lo_i[:, 1], hi_i[:, 0] + K * lo_i[:, 1], lo_i[:, 0] + K * hi_i[:, 1], hi_i[:, 0] + K * hi_i[:, 1]], axis=1)
    return basis, widx


def _pseudo(pos_s, pos_d):
    rel = pos_d - pos_s
    scale = jnp.maximum(jnp.max(jnp.abs(rel)), 1e-12)
    return jnp.clip(rel / (2.0 * scale) + 0.5, 0.0, 1.0)


def _head_kernel(gm_ref, gc_ref, fcw_ref, fcb_ref, out_ref):
    gm = gm_ref[...]
    gc = jnp.clip(gc_ref[...], 1.0)
    gm = gm / gc[:, None]
    logits = jnp.dot(gm, fcw_ref[...], preferred_element_type=jnp.float32) + fcb_ref[...][None, :]
    m = jnp.max(logits, axis=1, keepdims=True)
    z = logits - m
    lse = jnp.log(jnp.sum(jnp.exp(z), axis=1, keepdims=True))
    out_ref[...] = z - lse


def _head(gm, gc, fc_w, fc_b):
    g = gm.shape[0]
    return pl.pallas_call(
        _head_kernel,
        out_shape=jax.ShapeDtypeStruct((g, fc_w.shape[1]), jnp.float32),
    )(gm, gc, fc_w, fc_b)


def _edge_weights(sv, dv, valid_prev, nseg):
    valid = valid_prev & (sv != dv)
    key = jnp.where(valid, sv * nseg + dv, 0)
    cnt = jnp.zeros((nseg * nseg,), jnp.int32).at[key].add(valid.astype(jnp.int32))
    mult = cnt[key]
    ew = jnp.where(valid, 1.0 / jnp.maximum(mult, 1).astype(jnp.float32), 0.0)
    return ew, valid


def _spline_agg(feat_src, ew, dvox, basis, widx, W, nseg):
    fin = feat_src.shape[1]
    fout = W.shape[2]
    A = jnp.zeros((nseg * 25, fin), jnp.float32)
    for s in range(4):
        idx = dvox * 25 + widx[:, s]
        A = A.at[idx].add((ew * basis[:, s])[:, None] * feat_src)
    agg = A.reshape(nseg, 25 * fin) @ W.reshape(25 * fin, fout)
    deg = jax.ops.segment_sum(ew, dvox, num_segments=nseg)
    return agg / jnp.clip(deg, 1.0)[:, None]


def kernel(x, edge_index, pos, batch, W1, root1, b1, W2, root2, b2, W3, root3, b3, fc_w, fc_b):
    n = x.shape[0]
    src, dst = edge_index[0], edge_index[1]
    batch = batch.astype(jnp.int32)

    pseudo1 = _pseudo(pos[src], pos[dst])
    basis1, widx1 = _basis_widx(pseudo1)
    ew1 = jnp.ones((src.shape[0],), jnp.float32)
    agg1 = _spline_agg(x[src], ew1, dst, basis1, widx1, W1, n)
    h1 = jax.nn.elu(agg1 + x @ root1 + b1)

    n2 = 64 * 81
    g1 = jnp.floor(pos / 5.0).astype(jnp.int32)
    vox1 = batch * 81 + g1[:, 1] * 9 + g1[:, 0]
    ones_n = jnp.ones((n,), jnp.float32)
    cnt1 = jax.ops.segment_sum(ones_n, vox1, num_segments=n2)
    inv_cnt1 = 1.0 / jnp.maximum(cnt1, 1.0)
    pos2 = jax.ops.segment_sum(pos, vox1, num_segments=n2) * inv_cnt1[:, None]
    h2in = jax.ops.segment_sum(h1, vox1, num_segments=n2) * inv_cnt1[:, None]
    nv2 = cnt1 > 0.0
    batch2 = jnp.arange(n2, dtype=jnp.int32) // 81
    s2, d2 = vox1[src], vox1[dst]

    ew2, valid2 = _edge_weights(s2, d2, jnp.ones_like(s2, bool), n2)
    pseudo2 = _pseudo(pos2[s2], pos2[d2])
    basis2, widx2 = _basis_widx(pseudo2)
    agg2 = _spline_agg(h2in[s2], ew2, d2, basis2, widx2, W2, n2)
    h2 = jax.nn.elu(agg2 + h2in @ root2 + b2)

    n3 = 64 * 36
    g2 = jnp.floor(pos2 / 7.0).astype(jnp.int32)
    vox2 = batch2 * 36 + g2[:, 1] * 6 + g2[:, 0]
    w2 = nv2.astype(jnp.float32)
    cnt2 = jax.ops.segment_sum(w2, vox2, num_segments=n3)
    inv_cnt2 = 1.0 / jnp.maximum(cnt2, 1.0)
    pos3 = jax.ops.segment_sum(pos2 * w2[:, None], vox2, num_segments=n3) * inv_cnt2[:, None]
    h3in = jax.ops.segment_sum(h2 * w2[:, None], vox2, num_segments=n3) * inv_cnt2[:, None]
    nv3 = cnt2 > 0.0
    batch3 = jnp.arange(n3, dtype=jnp.int32) // 36
    s3, d3 = vox2[s2], vox2[d2]

    ew3, valid3 = _edge_weights(s3, d3, valid2, n3)
    pseudo3 = _pseudo(pos3[s3], pos3[d3])
    basis3, widx3 = _basis_widx(pseudo3)
    agg3 = _spline_agg(h3in[s3], ew3, d3, basis3, widx3, W3, n3)
    h3 = jax.nn.elu(agg3 + h3in @ root3 + b3)

    g = 64
    vm = nv3.astype(jnp.float32)
    gc = jax.ops.segment_sum(vm, batch3, num_segments=g)
    gm = jax.ops.segment_sum(h3 * vm[:, None], batch3, num_segments=g)
    out = _head(gm, gc, fc_w, fc_b)

    d = pos[:8].reshape(16)
    for _ in range(6):
        d = _sc_probe(d)
    return out * (1.0 + jnp.where(jnp.abs(d[0]) > 1e30, 1e-6, 0.0))


# speedup vs baseline: 1.2574x; 1.2574x over previous
import functools

import jax
import jax.numpy as jnp
from jax.experimental import pallas as pl

K = 5


def _basis_widx(pseudo):
    p = jnp.clip(pseudo, 0.0, 1.0) * (K - 1)
    lo = jnp.floor(p)
    frac = p - lo
    lo_i = jnp.clip(lo.astype(jnp.int32), 0, K - 1)
    hi_i = jnp.clip(lo_i + 1, 0, K - 1)
    b0 = 1.0 - frac
    b1 = frac
    basis = jnp.stack([b0[:, 0] * b0[:, 1], b1[:, 0] * b0[:, 1], b0[:, 0] * b1[:, 1], b1[:, 0] * b1[:, 1]], axis=1)
    widx = jnp.stack([lo_i[:, 0] + K * lo_i[:, 1], hi_i[:, 0] + K * lo_i[:, 1], lo_i[:, 0] + K * hi_i[:, 1], hi_i[:, 0] + K * hi_i[:, 1]], axis=1)
    return basis, widx


def _pseudo(pos_s, pos_d):
    rel = pos_d - pos_s
    scale = jnp.maximum(jnp.max(jnp.abs(rel)), 1e-12)
    return jnp.clip(rel / (2.0 * scale) + 0.5, 0.0, 1.0)


def _mm_kernel(x_ref, w_ref, o_ref):
    o_ref[...] = jnp.dot(x_ref[...], w_ref[...], preferred_element_type=jnp.float32)


def _xw_matmul(x, W, block_rows):
    n, fin = x.shape
    fout = W.shape[2]
    wt = W.transpose(1, 0, 2).reshape(fin, 25 * fout)
    grid = n // block_rows
    return pl.pallas_call(
        _mm_kernel,
        grid=(grid,),
        in_specs=[
            pl.BlockSpec((block_rows, fin), lambda i: (i, 0)),
            pl.BlockSpec((fin, 25 * fout), lambda i: (0, 0)),
        ],
        out_specs=pl.BlockSpec((block_rows, 25 * fout), lambda i: (i, 0)),
        out_shape=jax.ShapeDtypeStruct((n, 25 * fout), jnp.float32),
    )(x, wt)


def _spline_conv(x, xw, src, dst, pseudo, fout, root, bias, n, ew):
    basis, widx = _basis_widx(pseudo)
    xwr = xw.reshape(n * 25, fout)
    msg = jnp.zeros((src.shape[0], fout), jnp.float32)
    for s in range(4):
        msg = msg + (ew * basis[:, s])[:, None] * xwr[src * 25 + widx[:, s]]
    agg = jax.ops.segment_sum(msg, dst, num_segments=n)
    deg = jax.ops.segment_sum(ew, dst, num_segments=n)
    agg = agg / jnp.clip(deg, 1.0)[:, None]
    return agg + x @ root + bias


def _edge_weights(sv, dv, valid_prev, nseg):
    valid = valid_prev & (sv != dv)
    key = jnp.where(valid, sv * nseg + dv, 0)
    cnt = jnp.zeros((nseg * nseg,), jnp.int32).at[key].add(valid.astype(jnp.int32))
    mult = cnt[key]
    ew = jnp.where(valid, 1.0 / jnp.maximum(mult, 1).astype(jnp.float32), 0.0)
    return ew, valid


def _head_kernel(h_ref, vm_ref, fcw_ref, fcb_ref, out_ref):
    h = h_ref[...]
    vm = vm_ref[...]
    hw = h * vm
    gm = jnp.sum(hw.reshape(64, 36, h.shape[1]), axis=1)
    gc = jnp.clip(jnp.sum(vm.reshape(64, 36), axis=1), 1.0)
    gm = gm / gc[:, None]
    logits = jnp.dot(gm, fcw_ref[...], preferred_element_type=jnp.float32) + fcb_ref[...][None, :]
    m = jnp.max(logits, axis=1, keepdims=True)
    z = logits - m
    lse = jnp.log(jnp.sum(jnp.exp(z), axis=1, keepdims=True))
    out_ref[...] = z - lse


def _head(h3, vm, fc_w, fc_b):
    return pl.pallas_call(
        _head_kernel,
        out_shape=jax.ShapeDtypeStruct((64, fc_w.shape[1]), jnp.float32),
    )(h3, vm[:, None], fc_w, fc_b)


def kernel(x, edge_index, pos, batch, W1, root1, b1, W2, root2, b2, W3, root3, b3, fc_w, fc_b):
    n = x.shape[0]
    src, dst = edge_index[0].astype(jnp.int32), edge_index[1].astype(jnp.int32)
    batch = batch.astype(jnp.int32)
    e = src.shape[0]

    pseudo1 = _pseudo(pos[src], pos[dst])
    ew1 = jnp.ones((e,), jnp.float32)
    xw1 = _xw_matmul(x, W1, 400)
    h1 = jax.nn.elu(_spline_conv(x, xw1, src, dst, pseudo1, 32, root1, b1, n, ew1))

    n2 = 64 * 81
    g1 = jnp.floor(pos / 5.0).astype(jnp.int32)
    vox1 = batch * 81 + g1[:, 1] * 9 + g1[:, 0]
    feats1 = jnp.concatenate([h1, pos, jnp.ones((n, 1), jnp.float32)], axis=1)
    sums1 = jax.ops.segment_sum(feats1, vox1, num_segments=n2)
    cnt1 = sums1[:, 34]
    inv_cnt1 = 1.0 / jnp.maximum(cnt1, 1.0)
    h2in = sums1[:, :32] * inv_cnt1[:, None]
    pos2 = sums1[:, 32:34] * inv_cnt1[:, None]
    nv2 = cnt1 > 0.0
    batch2 = jnp.arange(n2, dtype=jnp.int32) // 81
    s2, d2 = vox1[src], vox1[dst]

    ew2, valid2 = _edge_weights(s2, d2, jnp.ones_like(s2, dtype=bool), n2)
    pseudo2 = _pseudo(pos2[s2], pos2[d2])
    xw2 = _xw_matmul(h2in, W2, 576)
    h2 = jax.nn.elu(_spline_conv(h2in, xw2, s2, d2, pseudo2, 64, root2, b2, n2, ew2))

    n3 = 64 * 36
    g2 = jnp.floor(pos2 / 7.0).astype(jnp.int32)
    vox2 = batch2 * 36 + g2[:, 1] * 6 + g2[:, 0]
    w2 = nv2.astype(jnp.float32)
    feats2 = jnp.concatenate([h2, pos2, jnp.ones((n2, 1), jnp.float32)], axis=1) * w2[:, None]
    sums2 = jax.ops.segment_sum(feats2, vox2, num_segments=n3)
    cnt2 = sums2[:, 66]
    inv_cnt2 = 1.0 / jnp.maximum(cnt2, 1.0)
    h3in = sums2[:, :64] * inv_cnt2[:, None]
    pos3 = sums2[:, 64:66] * inv_cnt2[:, None]
    nv3 = cnt2 > 0.0
    s3, d3 = vox2[s2], vox2[d2]

    ew3, valid3 = _edge_weights(s3, d3, valid2, n3)
    pseudo3 = _pseudo(pos3[s3], pos3[d3])
    xw3 = _xw_matmul(h3in, W3, 576)
    h3 = jax.nn.elu(_spline_conv(h3in, xw3, s3, d3, pseudo3, 64, root3, b3, n3, ew3))

    return _head(h3, nv3.astype(jnp.float32), fc_w, fc_b)
